# baseline (speedup 1.0000x reference)
import functools

import jax
import jax.numpy as jnp
from jax import lax
from jax.experimental import pallas as pl
from jax.experimental.pallas import tpu as pltpu
from jax.experimental.pallas import tpu_sc as plsc

B, C, H, W = 8, 8, 512, 512
NC, NS, L = 2, 16, 16
NW = NC * NS

SC_ROWS = 224
TC_ROWS = H - SC_ROWS

WROWS = SC_ROWS // (NW // B)
CROWS = 8
CHUNK = CROWS * W
NCH = WROWS // CROWS
GROUP = 8
NGRP = CHUNK // (GROUP * L)

_mesh = plsc.VectorSubcoreMesh(
    core_axis_name="c", subcore_axis_name="s", num_cores=NC, num_subcores=NS)


@functools.partial(
    pl.kernel,
    out_type=jax.ShapeDtypeStruct((NW, 32, L), jnp.int32),
    mesh=_mesh,
    scratch_types=[
        pltpu.VMEM((2, C, CROWS, W), jnp.float32),
        pltpu.VMEM((2, CROWS, W), jnp.int32),
        pltpu.VMEM((3 * C + 6, L), jnp.int32),
        pltpu.VMEM((32, L), jnp.int32),
        pltpu.SemaphoreType.DMA,
        pltpu.SemaphoreType.DMA,
    ],
)
def _dice_sc(logits_hbm, target_hbm, out_hbm, xbuf, tbuf, wacc, res,
             sem0, sem1):
    cid = lax.axis_index("c")
    sid = lax.axis_index("s")
    wid = sid * NC + cid
    b = wid % B
    wrow = TC_ROWS + (wid // B) * WROWS

    zero = jnp.zeros((L,), jnp.int32)
    one = jnp.full((L,), 1, jnp.int32)
    eightv = jnp.full((L,), C, jnp.int32)

    def fire(g, buf, sem):
        row = wrow + g * CROWS
        for cls in range(C):
            pltpu.async_copy(
                logits_hbm.at[b, cls, pl.ds(row, CROWS)],
                xbuf.at[buf, cls], sem)
        pltpu.async_copy(target_hbm.at[b, pl.ds(row, CROWS)],
                         tbuf.at[buf], sem)

    def drain(buf, sem):
        for cls in range(C):
            pltpu.make_async_copy(
                logits_hbm.at[b, cls, pl.ds(wrow, CROWS)],
                xbuf.at[buf, cls], sem).wait()
        pltpu.make_async_copy(target_hbm.at[b, pl.ds(wrow, CROWS)],
                              tbuf.at[buf], sem).wait()

    vec_per_row = W // L
    grp_per_row = vec_per_row // GROUP

    onehot = [jnp.full((L,), 1 << (4 * cls), jnp.int32) for cls in range(C)]
    bytemask = jnp.full((L,), 0x0F0F0F0F, jnp.int32)

    MID = 3 * C

    def compute(buf):
        def group(j, carry):
            acc_t = zero
            acc_i = zero
            acc_p = zero
            r = j // grp_per_row
            q = j % grp_per_row
            for s in range(GROUP):
                idx = (q * GROUP + s) * L
                xs = [xbuf[buf, cls, r, pl.ds(idx, L)] for cls in range(C)]
                t = tbuf[buf, r, pl.ds(idx, L)]
                ms, bits = list(xs), list(onehot)
                while len(ms) > 1:
                    nm, nb = [], []
                    for p in range(0, len(ms), 2):
                        keep = ms[p] >= ms[p + 1]
                        nm.append(jnp.maximum(ms[p], ms[p + 1]))
                        nb.append(jnp.where(keep, bits[p], bits[p + 1]))
                    ms, bits = nm, nb
                bit_p = bits[0]
                bit_t = one << (t << 2)
                acc_t = acc_t + bit_t
                acc_i = acc_i + jnp.where(bit_p == bit_t, bit_t, zero)
                acc_p = acc_p + bit_p
            for k, acc in ((0, acc_t), (1, acc_i), (2, acc_p)):
                wacc[MID + 2 * k] = wacc[MID + 2 * k] + (acc & bytemask)
                wacc[MID + 2 * k + 1] = (
                    wacc[MID + 2 * k + 1] + ((acc >> 4) & bytemask))
            return carry

        for half in range(2):
            for k in range(6):
                wacc[MID + k] = zero
            lax.fori_loop(
                0, NGRP // 2, lambda j, c, _h=half: group(j + _h * (NGRP // 2), c),
                0)
            b255 = jnp.full((L,), 255, jnp.int32)
            for stat in range(3):
                m0 = wacc[MID + 2 * stat]
                m1 = wacc[MID + 2 * stat + 1]
                for k in range(4):
                    sh = 8 * k
                    wacc[stat * C + 2 * k] = (
                        wacc[stat * C + 2 * k] + ((m0 >> sh) & b255))
                    wacc[stat * C + 2 * k + 1] = (
                        wacc[stat * C + 2 * k + 1] + ((m1 >> sh) & b255))

    for r in range(3 * C):
        wacc[r] = zero

    fire(0, 0, sem0)
    fire(1, 1, sem1)

    def outer(i, carry):
        g0 = i * 2

        drain(0, sem0)
        compute(0)

        @pl.when(g0 + 2 < NCH)
        def _():
            fire(g0 + 2, 0, sem0)

        drain(1, sem1)
        compute(1)

        @pl.when(g0 + 3 < NCH)
        def _():
            fire(g0 + 3, 1, sem1)

        return carry

    lax.fori_loop(0, NCH // 2, outer, 0)

    if NCH % 2:
        drain(0, sem0)
        compute(0)

    for r in range(3 * C):
        res[r] = wacc[r]
    for r in range(3 * C, 32):
        res[r] = zero
    pltpu.sync_copy(res, out_hbm.at[wid])


RB = 96
NK = TC_ROWS // RB
HALVINGS = 3 if RB % 64 == 0 else 2


def _dice_tc_body(logits_ref, target_ref, stats_ref, acc_ref):
    b = pl.program_id(0)
    k = pl.program_id(1)

    @pl.when(k == 0)
    def _init():
        acc_ref[...] = jnp.zeros_like(acc_ref)

    x = logits_ref[0]
    t = target_ref[0]

    m = x[0]
    for c in range(1, C):
        m = jnp.maximum(m, x[c])

    pred = jnp.full(t.shape, C, jnp.int32)
    for c in range(C - 1, -1, -1):
        pred = jnp.where(x[c] == m, c, pred)

    bit_t = jnp.left_shift(1, t << 2)
    bit_p = jnp.left_shift(1, pred << 2)
    bit_i = jnp.where(pred == t, bit_t, 0)

    for stat, bits in ((0, bit_t), (1, bit_i), (2, bit_p)):
        a = bits
        r = RB
        for _ in range(HALVINGS):
            r //= 2
            a = a[:r] + a[r:]
        for c in range(C):
            e = (a >> (4 * c)) & 15
            acc_ref[pl.ds(stat * C + c, 1)] += jnp.sum(
                e, axis=0, keepdims=True)

    @pl.when(k == NK - 1)
    def _flush():
        for stat in range(3):
            for c in range(C):
                stats_ref[stat, b, c] = jnp.sum(
                    acc_ref[stat * C + c]).astype(jnp.float32)


def _dice_tc(logits, target):
    return pl.pallas_call(
        _dice_tc_body,
        grid=(B, NK),
        in_specs=[
            pl.BlockSpec((1, C, RB, W), lambda b, k: (b, 0, k, 0)),
            pl.BlockSpec((1, RB, W), lambda b, k: (b, k, 0)),
        ],
        out_specs=pl.BlockSpec(
            (3, B, C), lambda b, k: (0, 0, 0), memory_space=pltpu.SMEM),
        out_shape=jax.ShapeDtypeStruct((3, B, C), jnp.float32),
        scratch_shapes=[
            pltpu.VMEM((3 * C, W), jnp.int32),
        ],
    )(logits, target)


def kernel(logits, target):
    tc_stats = _dice_tc(logits, target)
    sc_parts = _dice_sc(logits, target)

    counts = sc_parts[:, :3 * C, :].sum(axis=2).reshape(
        NW // B, B, 3, C).sum(axis=0).astype(jnp.float32)
    sc_s1 = counts[:, 0, :] + counts[:, 2, :]
    sc_i = counts[:, 1, :]

    s1 = tc_stats[0] + tc_stats[2] + sc_s1
    si = tc_stats[1] + sc_i
    union = s1 - si
    return jnp.mean(1.0 - (si + 1.0) / (union + 1.0))


# speedup vs baseline: 1.0321x; 1.0321x over previous
import functools

import jax
import jax.numpy as jnp
from jax import lax
from jax.experimental import pallas as pl
from jax.experimental.pallas import tpu as pltpu
from jax.experimental.pallas import tpu_sc as plsc

B, C, H, W = 8, 8, 512, 512
NC, NS, L = 2, 16, 16
NW = NC * NS

SC_ROWS = 256
TC_ROWS = H - SC_ROWS

WROWS = SC_ROWS // (NW // B)
CROWS = 8
CHUNK = CROWS * W
NCH = WROWS // CROWS
GROUP = 8
NGRP = CHUNK // (GROUP * L)

_mesh = plsc.VectorSubcoreMesh(
    core_axis_name="c", subcore_axis_name="s", num_cores=NC, num_subcores=NS)


@functools.partial(
    pl.kernel,
    out_type=jax.ShapeDtypeStruct((NW, 32, L), jnp.int32),
    mesh=_mesh,
    scratch_types=[
        pltpu.VMEM((2, C, CROWS, W), jnp.float32),
        pltpu.VMEM((2, CROWS, W), jnp.int32),
        pltpu.VMEM((3 * C + 6, L), jnp.int32),
        pltpu.VMEM((32, L), jnp.int32),
        pltpu.SemaphoreType.DMA,
        pltpu.SemaphoreType.DMA,
    ],
)
def _dice_sc(logits_hbm, target_hbm, out_hbm, xbuf, tbuf, wacc, res,
             sem0, sem1):
    cid = lax.axis_index("c")
    sid = lax.axis_index("s")
    wid = sid * NC + cid
    b = wid % B
    wrow = TC_ROWS + (wid // B) * WROWS

    zero = jnp.zeros((L,), jnp.int32)
    one = jnp.full((L,), 1, jnp.int32)
    eightv = jnp.full((L,), C, jnp.int32)

    def fire(g, buf, sem):
        row = wrow + g * CROWS
        for cls in range(C):
            pltpu.async_copy(
                logits_hbm.at[b, cls, pl.ds(row, CROWS)],
                xbuf.at[buf, cls], sem)
        pltpu.async_copy(target_hbm.at[b, pl.ds(row, CROWS)],
                         tbuf.at[buf], sem)

    def drain(buf, sem):
        for cls in range(C):
            pltpu.make_async_copy(
                logits_hbm.at[b, cls, pl.ds(wrow, CROWS)],
                xbuf.at[buf, cls], sem).wait()
        pltpu.make_async_copy(target_hbm.at[b, pl.ds(wrow, CROWS)],
                              tbuf.at[buf], sem).wait()

    vec_per_row = W // L
    grp_per_row = vec_per_row // GROUP

    onehot = [jnp.full((L,), 1 << (4 * cls), jnp.int32) for cls in range(C)]
    bytemask = jnp.full((L,), 0x0F0F0F0F, jnp.int32)

    MID = 3 * C

    def compute(buf):
        def group(j, carry):
            acc_t = zero
            acc_i = zero
            acc_p = zero
            r = j // grp_per_row
            q = j % grp_per_row
            for s in range(GROUP):
                idx = (q * GROUP + s) * L
                xs = [xbuf[buf, cls, r, pl.ds(idx, L)] for cls in range(C)]
                t = tbuf[buf, r, pl.ds(idx, L)]
                ms, bits = list(xs), list(onehot)
                while len(ms) > 1:
                    nm, nb = [], []
                    for p in range(0, len(ms), 2):
                        keep = ms[p] >= ms[p + 1]
                        nm.append(jnp.maximum(ms[p], ms[p + 1]))
                        nb.append(jnp.where(keep, bits[p], bits[p + 1]))
                    ms, bits = nm, nb
                bit_p = bits[0]
                bit_t = one << (t << 2)
                acc_t = acc_t + bit_t
                acc_i = acc_i + jnp.where(bit_p == bit_t, bit_t, zero)
                acc_p = acc_p + bit_p
            for k, acc in ((0, acc_t), (1, acc_i), (2, acc_p)):
                wacc[MID + 2 * k] = wacc[MID + 2 * k] + (acc & bytemask)
                wacc[MID + 2 * k + 1] = (
                    wacc[MID + 2 * k + 1] + ((acc >> 4) & bytemask))
            return carry

        for half in range(2):
            for k in range(6):
                wacc[MID + k] = zero
            lax.fori_loop(
                0, NGRP // 2, lambda j, c, _h=half: group(j + _h * (NGRP // 2), c),
                0)
            b255 = jnp.full((L,), 255, jnp.int32)
            for stat in range(3):
                m0 = wacc[MID + 2 * stat]
                m1 = wacc[MID + 2 * stat + 1]
                for k in range(4):
                    sh = 8 * k
                    wacc[stat * C + 2 * k] = (
                        wacc[stat * C + 2 * k] + ((m0 >> sh) & b255))
                    wacc[stat * C + 2 * k + 1] = (
                        wacc[stat * C + 2 * k + 1] + ((m1 >> sh) & b255))

    for r in range(3 * C):
        wacc[r] = zero

    fire(0, 0, sem0)
    fire(1, 1, sem1)

    def outer(i, carry):
        g0 = i * 2

        drain(0, sem0)
        compute(0)

        @pl.when(g0 + 2 < NCH)
        def _():
            fire(g0 + 2, 0, sem0)

        drain(1, sem1)
        compute(1)

        @pl.when(g0 + 3 < NCH)
        def _():
            fire(g0 + 3, 1, sem1)

        return carry

    lax.fori_loop(0, NCH // 2, outer, 0)

    if NCH % 2:
        drain(0, sem0)
        compute(0)

    for r in range(3 * C):
        res[r] = wacc[r]
    for r in range(3 * C, 32):
        res[r] = zero
    pltpu.sync_copy(res, out_hbm.at[wid])


RB = 256
NK = TC_ROWS // RB


def _dice_tc_body(logits_ref, target_ref, stats_ref, acc_ref):
    b = pl.program_id(0)
    k = pl.program_id(1)

    @pl.when(k == 0)
    def _init():
        acc_ref[...] = jnp.zeros_like(acc_ref)

    x = logits_ref[0]
    t = target_ref[0]

    m = x[0]
    for c in range(1, C):
        m = jnp.maximum(m, x[c])

    pred = jnp.full(t.shape, C, jnp.int32)
    for c in range(C - 1, -1, -1):
        pred = jnp.where(x[c] == m, c, pred)

    bit_t = jnp.left_shift(1, t << 2)
    bit_p = jnp.left_shift(1, pred << 2)
    bit_i = jnp.where(pred == t, bit_t, 0)

    for stat, bits in ((0, bit_t), (1, bit_i), (2, bit_p)):
        a = bits
        r = RB
        for _ in range(3):
            r //= 2
            a = a[:r] + a[r:]
        for c in range(C):
            e = (a >> (4 * c)) & 15
            acc_ref[pl.ds(stat * C + c, 1)] += jnp.sum(
                e, axis=0, keepdims=True)

    @pl.when(k == NK - 1)
    def _flush():
        for stat in range(3):
            for c in range(C):
                stats_ref[stat, b, c] = jnp.sum(
                    acc_ref[stat * C + c]).astype(jnp.float32)


def _dice_tc(logits, target):
    return pl.pallas_call(
        _dice_tc_body,
        grid=(B, NK),
        in_specs=[
            pl.BlockSpec((1, C, RB, W), lambda b, k: (b, 0, k, 0)),
            pl.BlockSpec((1, RB, W), lambda b, k: (b, k, 0)),
        ],
        out_specs=pl.BlockSpec(
            (3, B, C), lambda b, k: (0, 0, 0), memory_space=pltpu.SMEM),
        out_shape=jax.ShapeDtypeStruct((3, B, C), jnp.float32),
        scratch_shapes=[
            pltpu.VMEM((3 * C, W), jnp.int32),
        ],
    )(logits, target)


def kernel(logits, target):
    tc_stats = _dice_tc(logits, target)
    sc_parts = _dice_sc(logits, target)

    counts = sc_parts[:, :3 * C, :].sum(axis=2).reshape(
        NW // B, B, 3, C).sum(axis=0).astype(jnp.float32)
    sc_s1 = counts[:, 0, :] + counts[:, 2, :]
    sc_i = counts[:, 1, :]

    s1 = tc_stats[0] + tc_stats[2] + sc_s1
    si = tc_stats[1] + sc_i
    union = s1 - si
    return jnp.mean(1.0 - (si + 1.0) / (union + 1.0))


# speedup vs baseline: 1.0401x; 1.0077x over previous
import functools

import jax
import jax.numpy as jnp
from jax import lax
from jax.experimental import pallas as pl
from jax.experimental.pallas import tpu as pltpu
from jax.experimental.pallas import tpu_sc as plsc

B, C, H, W = 8, 8, 512, 512
NC, NS, L = 2, 16, 16
NW = NC * NS

SC_ROWS = 224
TC_ROWS = H - SC_ROWS

WROWS = SC_ROWS // (NW // B)
CROWS = 8
CHUNK = CROWS * W
NCH = WROWS // CROWS
GROUP = 8
NGRP = CHUNK // (GROUP * L)

_mesh = plsc.VectorSubcoreMesh(
    core_axis_name="c", subcore_axis_name="s", num_cores=NC, num_subcores=NS)


@functools.partial(
    pl.kernel,
    out_type=jax.ShapeDtypeStruct((NW, 32, L), jnp.int32),
    mesh=_mesh,
    scratch_types=[
        pltpu.VMEM((2, C, CROWS, W), jnp.float32),
        pltpu.VMEM((2, CROWS, W), jnp.int32),
        pltpu.VMEM((3 * C + 6, L), jnp.int32),
        pltpu.VMEM((32, L), jnp.int32),
        pltpu.SemaphoreType.DMA,
        pltpu.SemaphoreType.DMA,
    ],
)
def _dice_sc(logits_hbm, target_hbm, out_hbm, xbuf, tbuf, wacc, res,
             sem0, sem1):
    cid = lax.axis_index("c")
    sid = lax.axis_index("s")
    wid = sid * NC + cid
    b = wid % B
    wrow = TC_ROWS + (wid // B) * WROWS

    zero = jnp.zeros((L,), jnp.int32)
    one = jnp.full((L,), 1, jnp.int32)
    eightv = jnp.full((L,), C, jnp.int32)

    def fire(g, buf, sem):
        row = wrow + g * CROWS
        for cls in range(C):
            pltpu.async_copy(
                logits_hbm.at[b, cls, pl.ds(row, CROWS)],
                xbuf.at[buf, cls], sem)
        pltpu.async_copy(target_hbm.at[b, pl.ds(row, CROWS)],
                         tbuf.at[buf], sem)

    def drain(buf, sem):
        for cls in range(C):
            pltpu.make_async_copy(
                logits_hbm.at[b, cls, pl.ds(wrow, CROWS)],
                xbuf.at[buf, cls], sem).wait()
        pltpu.make_async_copy(target_hbm.at[b, pl.ds(wrow, CROWS)],
                              tbuf.at[buf], sem).wait()

    vec_per_row = W // L
    grp_per_row = vec_per_row // GROUP

    onehot = [jnp.full((L,), 1 << (4 * cls), jnp.int32) for cls in range(C)]
    bytemask = jnp.full((L,), 0x0F0F0F0F, jnp.int32)

    MID = 3 * C

    def compute(buf):
        def group(j, carry):
            acc_t = zero
            acc_i = zero
            acc_p = zero
            r = j // grp_per_row
            q = j % grp_per_row
            for s in range(GROUP):
                idx = (q * GROUP + s) * L
                xs = [xbuf[buf, cls, r, pl.ds(idx, L)] for cls in range(C)]
                t = tbuf[buf, r, pl.ds(idx, L)]
                ms, bits = list(xs), list(onehot)
                while len(ms) > 1:
                    nm, nb = [], []
                    for p in range(0, len(ms), 2):
                        keep = ms[p] >= ms[p + 1]
                        nm.append(jnp.maximum(ms[p], ms[p + 1]))
                        nb.append(jnp.where(keep, bits[p], bits[p + 1]))
                    ms, bits = nm, nb
                bit_p = bits[0]
                bit_t = one << (t << 2)
                acc_t = acc_t + bit_t
                acc_i = acc_i + jnp.where(bit_p == bit_t, bit_t, zero)
                acc_p = acc_p + bit_p
            for k, acc in ((0, acc_t), (1, acc_i), (2, acc_p)):
                wacc[MID + 2 * k] = wacc[MID + 2 * k] + (acc & bytemask)
                wacc[MID + 2 * k + 1] = (
                    wacc[MID + 2 * k + 1] + ((acc >> 4) & bytemask))
            return carry

        for half in range(2):
            for k in range(6):
                wacc[MID + k] = zero
            lax.fori_loop(
                0, NGRP // 2, lambda j, c, _h=half: group(j + _h * (NGRP // 2), c),
                0)
            b255 = jnp.full((L,), 255, jnp.int32)
            for stat in range(3):
                m0 = wacc[MID + 2 * stat]
                m1 = wacc[MID + 2 * stat + 1]
                for k in range(4):
                    sh = 8 * k
                    wacc[stat * C + 2 * k] = (
                        wacc[stat * C + 2 * k] + ((m0 >> sh) & b255))
                    wacc[stat * C + 2 * k + 1] = (
                        wacc[stat * C + 2 * k + 1] + ((m1 >> sh) & b255))

    for r in range(3 * C):
        wacc[r] = zero

    fire(0, 0, sem0)
    fire(1, 1, sem1)

    def outer(i, carry):
        g0 = i * 2

        drain(0, sem0)
        compute(0)

        @pl.when(g0 + 2 < NCH)
        def _():
            fire(g0 + 2, 0, sem0)

        drain(1, sem1)
        compute(1)

        @pl.when(g0 + 3 < NCH)
        def _():
            fire(g0 + 3, 1, sem1)

        return carry

    lax.fori_loop(0, NCH // 2, outer, 0)

    if NCH % 2:
        drain(0, sem0)
        compute(0)

    for r in range(3 * C):
        res[r] = wacc[r]
    for r in range(3 * C, 32):
        res[r] = zero
    pltpu.sync_copy(res, out_hbm.at[wid])


RB = 288
NK = TC_ROWS // RB


def _dice_tc_body(logits_ref, target_ref, stats_ref, acc_ref):
    b = pl.program_id(0)
    k = pl.program_id(1)

    @pl.when(k == 0)
    def _init():
        acc_ref[...] = jnp.zeros_like(acc_ref)

    x = logits_ref[0]
    t = target_ref[0]

    m = x[0]
    for c in range(1, C):
        m = jnp.maximum(m, x[c])

    pred = jnp.full(t.shape, C, jnp.int32)
    for c in range(C - 1, -1, -1):
        pred = jnp.where(x[c] == m, c, pred)

    bit_t = jnp.left_shift(1, t << 2)
    bit_p = jnp.left_shift(1, pred << 2)
    bit_i = jnp.where(pred == t, bit_t, 0)

    for stat, bits in ((0, bit_t), (1, bit_i), (2, bit_p)):
        a = bits
        r = RB
        for _ in range(3):
            r //= 2
            a = a[:r] + a[r:]
        for c in range(C):
            e = (a >> (4 * c)) & 15
            acc_ref[pl.ds(stat * C + c, 1)] += jnp.sum(
                e, axis=0, keepdims=True)

    @pl.when(k == NK - 1)
    def _flush():
        for stat in range(3):
            for c in range(C):
                stats_ref[stat, b, c] = jnp.sum(
                    acc_ref[stat * C + c]).astype(jnp.float32)


def _dice_tc(logits, target):
    return pl.pallas_call(
        _dice_tc_body,
        grid=(B, NK),
        in_specs=[
            pl.BlockSpec((1, C, RB, W), lambda b, k: (b, 0, k, 0)),
            pl.BlockSpec((1, RB, W), lambda b, k: (b, k, 0)),
        ],
        out_specs=pl.BlockSpec(
            (3, B, C), lambda b, k: (0, 0, 0), memory_space=pltpu.SMEM),
        out_shape=jax.ShapeDtypeStruct((3, B, C), jnp.float32),
        scratch_shapes=[
            pltpu.VMEM((3 * C, W), jnp.int32),
        ],
    )(logits, target)


def kernel(logits, target):
    tc_stats = _dice_tc(logits, target)
    sc_parts = _dice_sc(logits, target)

    counts = sc_parts[:, :3 * C, :].sum(axis=2).reshape(
        NW // B, B, 3, C).sum(axis=0).astype(jnp.float32)
    sc_s1 = counts[:, 0, :] + counts[:, 2, :]
    sc_i = counts[:, 1, :]

    s1 = tc_stats[0] + tc_stats[2] + sc_s1
    si = tc_stats[1] + sc_i
    union = s1 - si
    return jnp.mean(1.0 - (si + 1.0) / (union + 1.0))
